# baseline (speedup 1.0000x reference)
import functools

import jax
import jax.numpy as jnp
from jax import lax
from jax.experimental import pallas as pl
from jax.experimental.pallas import tpu as pltpu
from jax.experimental.pallas import tpu_sc as plsc

_N_GROUP = 4
_TOPK_GROUP = 2
_TOP_K = 2
_SCALE = 2.5

_NC = 2
_NS = 16

_BM = 256


def _sigmoid(x):
    return 1.0 / (1.0 + jnp.exp(-x))


def _dot(a, b):
    return lax.dot_general(a, b, (((1,), (0,)), ((), ())),
                           preferred_element_type=jnp.float32)


def _route_body(xb_ref, gw_ref, bias_ref, eidx_ref, wts_ref):
    i32 = jnp.int32
    f32 = jnp.float32
    e = gw_ref.shape[0]
    t = xb_ref.shape[0]
    logits = lax.dot_general(gw_ref[...], xb_ref[...],
                             (((1,), (1,)), ((), ())),
                             preferred_element_type=f32)
    scores = _sigmoid(logits)
    swb = scores + bias_ref[...]
    rows = [swb[j:j + 1, :] for j in range(e)]
    gsz = e // _N_GROUP
    grp = [rows[g * gsz] + rows[g * gsz + 1] for g in range(_N_GROUP)]
    selg = []
    for j in range(_N_GROUP):
        r = jnp.zeros_like(grp[j])
        for k in range(_N_GROUP):
            if k == j:
                continue
            beats = (grp[k] >= grp[j]) if k < j else (grp[k] > grp[j])
            r = r + beats.astype(f32)
        selg.append(r < float(_TOPK_GROUP))
    vals = [jnp.where(selg[j // gsz], rows[j], 0.0) for j in range(e)]
    sel = []
    for j in range(e):
        r = jnp.zeros_like(vals[j])
        for k in range(e):
            if k == j:
                continue
            beats = (vals[k] >= vals[j]) if k < j else (vals[k] > vals[j])
            r = r + beats.astype(f32)
        sel.append(r < float(_TOP_K))
    sm = [jnp.where(sel[j], scores[j:j + 1, :], 0.0) for j in range(e)]
    ssum = sm[0] + sm[1]
    for j in range(2, e):
        ssum = ssum + sm[j]
    ssum = ssum + 1e-20
    cw = [s / ssum * _SCALE for s in sm]
    emin = jnp.full((1, t), e + 1, i32)
    emax = jnp.full((1, t), -1, i32)
    for j in range(e):
        jf = jnp.full((1, t), j, i32)
        emin = jnp.where(sel[j], jnp.minimum(emin, jf), emin)
        emax = jnp.where(sel[j], jnp.maximum(emax, jf), emax)
    w0 = jnp.zeros((1, t), f32)
    w1 = jnp.zeros((1, t), f32)
    for j in range(e):
        jf = jnp.full((1, t), j, i32)
        w0 = w0 + jnp.where(emin == jf, cw[j], 0.0)
        w1 = w1 + jnp.where(emax == jf, cw[j], 0.0)
    eidx_ref[...] = jnp.concatenate([emin, emax] * (e // 2), axis=0)
    wts_ref[...] = jnp.concatenate([w0, w1] * (e // 2), axis=0)


_CUT = 5


def _make_sort_body(T, E, BM, NP):
    A = 2 * T
    CH = A // _NS
    NV = CH // 16
    TOKPF = NP // _NS
    W8PF = NP * E // _NS
    SH = BM.bit_length() - 1
    i32 = jnp.int32
    f32 = jnp.float32

    def body(eidx_hbm, wts_hbm,
             pos_hbm, tok_hbm, w8_hbm, te_hbm, na_hbm,
             ev, wv, posv, idx8v, tokv, hv, tblv, basev, stv, ntv,
             tev, nav, tmpv, zbuf, pbuf, tbl_sh, sem):
        del sem
        cid = lax.axis_index("c")
        tid = lax.axis_index("s")
        iota = lax.iota(i32, 16)

        @pl.when(cid == 0)
        def _work():
            row = jnp.where(tid < _NS // 2, 0, 1)
            col0 = jnp.where(tid < _NS // 2, tid * CH, (tid - _NS // 2) * CH)
            pltpu.sync_copy(eidx_hbm.at[row, pl.ds(col0, CH)], ev)
            pltpu.sync_copy(wts_hbm.at[row, pl.ds(col0, CH)], wv)

            if _CUT <= 0:
                pltpu.sync_copy(ev, pos_hbm.at[pl.ds(tid * CH, CH)])
                return

            def _eqz(x):
                return 1 - (jnp.right_shift(x | (0 - x), 31) & 1)

            def _incl_prefix(x):
                p = x
                for s in (1, 2, 4, 8):
                    tmpv[...] = p
                    g = plsc.load_gather(tmpv, [jnp.maximum(iota - s, 0)])
                    m = 1 + jnp.right_shift(iota - s, 31)
                    p = p + g * m
                return p

            def _splat(x, lane):
                tmpv[...] = x
                return plsc.load_gather(tmpv, [iota * 0 + lane])

            cs = [jnp.zeros((16,), i32) for _ in range(E)]
            for v in range(NV):
                sidx = iota * 16 + v
                evec = plsc.load_gather(ev, [sidx])
                r = jnp.zeros((16,), i32)
                for ex in range(E):
                    eq = _eqz(evec - ex)
                    r = r + eq * cs[ex]
                    cs[ex] = cs[ex] + eq
                plsc.store_scatter(posv, [sidx], r)

            if _CUT <= 1:
                pltpu.sync_copy(posv, pos_hbm.at[pl.ds(tid * CH, CH)])
                return

            exq = []
            hvec = jnp.zeros((16,), i32)
            for ex in range(E):
                inc = _incl_prefix(cs[ex])
                exq.append(inc - cs[ex])
                hvec = hvec + _eqz(iota - ex) * _splat(inc, 15)
            hv[...] = hvec
            pltpu.sync_copy(hv, tbl_sh.at[pl.ds(tid * 16, 16)])
            plsc.subcore_barrier()
            pltpu.sync_copy(tbl_sh, tblv)

            total = jnp.zeros((16,), i32)
            mybase = jnp.zeros((16,), i32)
            for rr in range(_NS):
                rowv = tblv[pl.ds(rr * 16, 16)]
                total = total + rowv
                lt = jnp.right_shift(rr - tid, 31) & 1
                mybase = mybase + rowv * lt

            padded = (total + (BM - 1)) & (-BM)
            incp = _incl_prefix(padded)
            offs = incp - padded
            basev[...] = offs + mybase
            tile_start = jnp.right_shift(offs, SH)
            ntiles = jnp.right_shift(padded, SH)
            stv[...] = tile_start
            ntv[...] = ntiles
            nav[...] = jnp.right_shift(_splat(incp, 15), SH)

            if _CUT <= 2:
                pltpu.sync_copy(posv, pos_hbm.at[pl.ds(tid * CH, CH)])
                return

            for v in range(NV):
                sidx = iota * 16 + v
                evec = plsc.load_gather(ev, [sidx])
                r = plsc.load_gather(posv, [sidx])
                for ex in range(E):
                    r = r + _eqz(evec - ex) * exq[ex]
                p = r + plsc.load_gather(basev, [evec])
                plsc.store_scatter(posv, [sidx], p)
                plsc.store_scatter(idx8v, [sidx], p * E)
            pltpu.sync_copy(posv, pos_hbm.at[pl.ds(tid * CH, CH)])

            for v in range(TOKPF // 16):
                pbuf[pl.ds(v * 16, 16)] = (iota + (tid * TOKPF + v * 16)) & (T - 1)
            pltpu.sync_copy(pbuf, tok_hbm.at[pl.ds(tid * TOKPF, TOKPF)])
            zv = jnp.zeros((16,), f32)
            for v in range(W8PF // 16):
                zbuf[pl.ds(v * 16, 16)] = zv
            pltpu.sync_copy(zbuf, w8_hbm.at[pl.ds(tid * W8PF, W8PF)])

            plsc.subcore_barrier()

            if _CUT <= 3:
                return

            tb = jnp.where(tid < _NS // 2, tid * CH, (tid - _NS // 2) * CH)
            for v in range(NV):
                tokv[pl.ds(v * 16, 16)] = iota + (tb + v * 16)
            pltpu.sync_copy(tokv, tok_hbm.at[posv])
            pltpu.sync_copy(wv, w8_hbm.at[idx8v])

            if _CUT <= 4:
                return

            @pl.when(tid == 0)
            def _tile0():
                for half in range(2 * _NS // 16):
                    j = iota + 16 * half
                    acc = jnp.zeros((16,), i32)
                    for ex in range(E):
                        st = plsc.load_gather(stv, [iota * 0 + ex])
                        nt = plsc.load_gather(ntv, [iota * 0 + ex])
                        geq = 1 + jnp.right_shift(j - st, 31)
                        lt = 0 - jnp.right_shift(j - st - nt, 31)
                        acc = acc + geq * lt * ex
                    tev[pl.ds(16 * half, 16)] = acc
                pltpu.sync_copy(tev, te_hbm)
                pltpu.sync_copy(nav, na_hbm)

    return body


def _make_gather_body(NP, CHUNK):
    NW = _NC * _NS
    RPW = NP // NW

    def body(tok_hbm, x_hbm, xs_hbm, idxv, rowsv, sem):
        wid = lax.axis_index("s") * _NC + lax.axis_index("c")
        for c in range(RPW // CHUNK):
            base = wid * RPW + c * CHUNK
            pltpu.sync_copy(tok_hbm.at[pl.ds(base, CHUNK)], idxv)
            pltpu.async_copy(x_hbm.at[idxv], rowsv, sem).wait()
            pltpu.sync_copy(rowsv, xs_hbm.at[pl.ds(base, CHUNK)])

    return body


def _experts_body(te_ref, na_ref, xs_ref, wg_ref, wu_ref, wd_ref, w8_ref,
                  ys_ref):
    i = pl.program_id(0)

    @pl.when(i < na_ref[0])
    def _():
        xb = xs_ref[...].astype(jnp.bfloat16)
        h1 = _dot(xb, wg_ref[0])
        h2 = _dot(xb, wu_ref[0])
        act = (h1 * _sigmoid(h1)) * h2
        y = _dot(act.astype(jnp.bfloat16), wd_ref[0])
        wcol = jnp.sum(w8_ref[...], axis=1, keepdims=True)
        ys_ref[...] = wcol * y


def _shared_body(xb_ref, sg_ref, su_ref, sd_ref, sh_ref):
    xb = xb_ref[...]
    s1 = _dot(xb, sg_ref[...])
    s2 = _dot(xb, su_ref[...])
    sact = (s1 * _sigmoid(s1)) * s2
    sh_ref[...] = _dot(sact.astype(jnp.bfloat16), sd_ref[...])


def _make_combine_body(T, CHUNK):
    NW = _NC * _NS
    TPW = T // NW

    def body(ys_hbm, pos_hbm, g0_hbm, g1_hbm, p0v, p1v, r0v, r1v, sem):
        wid = lax.axis_index("s") * _NC + lax.axis_index("c")
        for c in range(TPW // CHUNK):
            base = wid * TPW + c * CHUNK
            pltpu.sync_copy(pos_hbm.at[pl.ds(base, CHUNK)], p0v)
            pltpu.sync_copy(pos_hbm.at[pl.ds(T + base, CHUNK)], p1v)
            pltpu.async_copy(ys_hbm.at[p0v], r0v, sem).wait()
            pltpu.async_copy(ys_hbm.at[p1v], r1v, sem).wait()
            pltpu.sync_copy(r0v, g0_hbm.at[pl.ds(base, CHUNK)])
            pltpu.sync_copy(r1v, g1_hbm.at[pl.ds(base, CHUNK)])

    return body


def _final_body(sh_ref, g0_ref, g1_ref, out_ref):
    out_ref[...] = sh_ref[...] + g0_ref[...] + g1_ref[...]


def kernel_staged(hidden_states, gate_weight, e_score_correction_bias,
                  w_gate, w_up, w_down, s_gate, s_up, s_down, stages=7):
    t, h = hidden_states.shape
    n_experts, _, ff = w_gate.shape
    sff = s_gate.shape[1]
    i32 = jnp.int32
    f32 = jnp.float32
    bf = jnp.bfloat16

    a = _TOP_K * t
    maxt = a // _BM + n_experts
    np_rows = maxt * _BM

    xb = hidden_states.astype(bf)
    gwb = gate_weight.astype(bf)
    bias2d = e_score_correction_bias.reshape(n_experts, 1)
    wgb = w_gate.astype(bf)
    wub = w_up.astype(bf)
    wdb = w_down.astype(bf)
    sgb = s_gate.astype(bf)
    sub = s_up.astype(bf)
    sdb = s_down.astype(bf)

    eidx, wts = pl.pallas_call(
        _route_body,
        out_shape=[jax.ShapeDtypeStruct((n_experts, t), i32),
                   jax.ShapeDtypeStruct((n_experts, t), f32)],
    )(xb, gwb, bias2d)
    if stages == 1:
        return eidx, wts

    mesh = plsc.VectorSubcoreMesh(core_axis_name="c", subcore_axis_name="s")
    ch = a // _NS
    sort_call = pl.kernel(
        _make_sort_body(t, n_experts, _BM, np_rows),
        out_type=[jax.ShapeDtypeStruct((a,), i32),
                  jax.ShapeDtypeStruct((np_rows,), i32),
                  jax.ShapeDtypeStruct((np_rows * n_experts,), f32),
                  jax.ShapeDtypeStruct((2 * _NS,), i32),
                  jax.ShapeDtypeStruct((16,), i32)],
        mesh=mesh,
        compiler_params=pltpu.CompilerParams(needs_layout_passes=False),
        scratch_types=[
            pltpu.VMEM((ch,), i32),
            pltpu.VMEM((ch,), f32),
            pltpu.VMEM((ch,), i32),
            pltpu.VMEM((ch,), i32),
            pltpu.VMEM((ch,), i32),
            pltpu.VMEM((16,), i32),
            pltpu.VMEM((16 * _NS,), i32),
            pltpu.VMEM((16,), i32),
            pltpu.VMEM((16,), i32),
            pltpu.VMEM((16,), i32),
            pltpu.VMEM((2 * _NS,), i32),
            pltpu.VMEM((16,), i32),
            pltpu.VMEM((16,), i32),
            pltpu.VMEM((np_rows * n_experts // _NS,), f32),
            pltpu.VMEM((np_rows // _NS,), i32),
            pltpu.VMEM_SHARED((16 * _NS,), i32),
            pltpu.SemaphoreType.DMA,
        ],
    )
    pos, tok, w8f, te, na = sort_call(eidx, wts)
    if stages == 2:
        return pos, tok, w8f, te, na

    gather_call = pl.kernel(
        _make_gather_body(np_rows, 48),
        out_type=jax.ShapeDtypeStruct((np_rows, h), f32),
        mesh=mesh,
        compiler_params=pltpu.CompilerParams(needs_layout_passes=False),
        scratch_types=[
            pltpu.VMEM((48,), i32),
            pltpu.VMEM((48, h), f32),
            pltpu.SemaphoreType.DMA,
        ],
    )
    xs = gather_call(tok, hidden_states)
    if stages == 3:
        return xs, pos, w8f, te, na

    w8 = w8f.reshape(np_rows, n_experts)
    grid_spec = pltpu.PrefetchScalarGridSpec(
        num_scalar_prefetch=2,
        grid=(maxt,),
        in_specs=[
            pl.BlockSpec((_BM, h),
                         lambda i, te, na: (jnp.minimum(i, na[0] - 1), 0)),
            pl.BlockSpec((1, h, ff),
                         lambda i, te, na: (te[jnp.minimum(i, na[0] - 1)], 0, 0)),
            pl.BlockSpec((1, h, ff),
                         lambda i, te, na: (te[jnp.minimum(i, na[0] - 1)], 0, 0)),
            pl.BlockSpec((1, ff, h),
                         lambda i, te, na: (te[jnp.minimum(i, na[0] - 1)], 0, 0)),
            pl.BlockSpec((_BM, n_experts),
                         lambda i, te, na: (jnp.minimum(i, na[0] - 1), 0)),
        ],
        out_specs=pl.BlockSpec((_BM, h), lambda i, te, na: (i, 0)),
    )
    ys = pl.pallas_call(
        _experts_body,
        grid_spec=grid_spec,
        out_shape=jax.ShapeDtypeStruct((np_rows, h), f32),
        compiler_params=pltpu.CompilerParams(
            dimension_semantics=("arbitrary",)),
    )(te, na, xs, wgb, wub, wdb, w8)
    if stages == 4:
        return ys, pos

    bts = 512
    sh = pl.pallas_call(
        _shared_body,
        grid=(t // bts,),
        in_specs=[
            pl.BlockSpec((bts, h), lambda i: (i, 0)),
            pl.BlockSpec((h, sff), lambda i: (0, 0)),
            pl.BlockSpec((h, sff), lambda i: (0, 0)),
            pl.BlockSpec((sff, h), lambda i: (0, 0)),
        ],
        out_specs=pl.BlockSpec((bts, h), lambda i: (i, 0)),
        out_shape=jax.ShapeDtypeStruct((t, h), f32),
    )(xb, sgb, sub, sdb)
    if stages == 5:
        return ys, pos, sh

    combine_call = pl.kernel(
        _make_combine_body(t, 32),
        out_type=[jax.ShapeDtypeStruct((t, h), f32),
                  jax.ShapeDtypeStruct((t, h), f32)],
        mesh=mesh,
        compiler_params=pltpu.CompilerParams(needs_layout_passes=False),
        scratch_types=[
            pltpu.VMEM((32,), i32),
            pltpu.VMEM((32,), i32),
            pltpu.VMEM((32, h), f32),
            pltpu.VMEM((32, h), f32),
            pltpu.SemaphoreType.DMA,
        ],
    )
    g0, g1 = combine_call(ys, pos)
    if stages == 6:
        return g0, g1, sh

    out = pl.pallas_call(
        _final_body,
        grid=(t // bts,),
        in_specs=[
            pl.BlockSpec((bts, h), lambda i: (i, 0)),
            pl.BlockSpec((bts, h), lambda i: (i, 0)),
            pl.BlockSpec((bts, h), lambda i: (i, 0)),
        ],
        out_specs=pl.BlockSpec((bts, h), lambda i: (i, 0)),
        out_shape=jax.ShapeDtypeStruct((t, h), f32),
    )(sh, g0, g1)
    return out


_STAGES = 7


def kernel(hidden_states, gate_weight, e_score_correction_bias,
           w_gate, w_up, w_down, s_gate, s_up, s_down):
    outs = kernel_staged(hidden_states, gate_weight, e_score_correction_bias,
                         w_gate, w_up, w_down, s_gate, s_up, s_down,
                         stages=_STAGES)
    if _STAGES == 7:
        return outs
    t, h = hidden_states.shape
    acc = 0.0
    for o in jax.tree.leaves(outs):
        acc = acc + o.ravel()[0].astype(jnp.float32)
    return jnp.zeros((t, h), jnp.float32) + acc
